# baseline (speedup 1.0000x reference)
import functools

import jax
import jax.numpy as jnp
from jax import lax
from jax.experimental import pallas as pl
from jax.experimental.pallas import tpu as pltpu
from jax.experimental.pallas import tpu_sc as plsc

NC = 2
NS = 16
L = 16
NW = NC * NS

B = 16384
D = 64
BPW = B // NW
CHUNK = 128
NCHUNK = BPW // CHUNK

MV_BLK = 20480


def _matvec_body(w_ref, t_ref, o_ref):
    o_ref[...] = lax.dot_general(
        w_ref[...], t_ref[...], (((1,), (0,)), ((), ())),
        preferred_element_type=jnp.float32).reshape(o_ref.shape)


def _scores(t_T, w_row):
    n = t_T.shape[1]
    blk = min(MV_BLK, n)
    grid = pl.cdiv(n, blk)
    out = pl.pallas_call(
        _matvec_body,
        grid=(grid,),
        in_specs=[
            pl.BlockSpec((1, D), lambda i: (0, 0)),
            pl.BlockSpec((D, blk), lambda i: (0, i)),
        ],
        out_specs=pl.BlockSpec((blk,), lambda i: (i,)),
        out_shape=jax.ShapeDtypeStruct((n,), jnp.float32),
    )(w_row, t_T)
    return out


def _matvec2_body(w_ref, tu_ref, ti_ref, ou_ref, oi_ref):
    ou_ref[...] = lax.dot_general(
        w_ref[0:1], tu_ref[...], (((1,), (0,)), ((), ())),
        preferred_element_type=jnp.float32).reshape(ou_ref.shape)
    oi_ref[...] = lax.dot_general(
        w_ref[1:2], ti_ref[...], (((1,), (0,)), ((), ())),
        preferred_element_type=jnp.float32).reshape(oi_ref.shape)


def _scores2(tu_T, ti_T, w2):
    n = tu_T.shape[1]
    blk = MV_BLK
    grid = pl.cdiv(n, blk)
    shape = jax.ShapeDtypeStruct((n,), jnp.float32)
    return pl.pallas_call(
        _matvec2_body,
        grid=(grid,),
        in_specs=[
            pl.BlockSpec((2, D), lambda i: (0, 0)),
            pl.BlockSpec((D, blk), lambda i: (0, i)),
            pl.BlockSpec((D, blk), lambda i: (0, i)),
        ],
        out_specs=[pl.BlockSpec((blk,), lambda i: (i,)),
                   pl.BlockSpec((blk,), lambda i: (i,))],
        out_shape=[shape, shape],
    )(w2, tu_T, ti_T)


@functools.cache
def _build_combine():
    mesh = plsc.VectorSubcoreMesh(core_axis_name="c", subcore_axis_name="s",
                                  num_cores=NC, num_subcores=NS)

    @functools.partial(
        pl.kernel,
        out_type=jax.ShapeDtypeStruct((B,), jnp.float32),
        mesh=mesh,
        scratch_types=[
            pltpu.VMEM((NCHUNK, CHUNK), jnp.int32),
            pltpu.VMEM((NCHUNK, CHUNK), jnp.int32),
            pltpu.VMEM((NCHUNK, CHUNK), jnp.int32),
            pltpu.VMEM((BPW,), jnp.float32),
            pltpu.VMEM((BPW,), jnp.float32),
            pltpu.VMEM((BPW,), jnp.float32),
            pltpu.VMEM((L,), jnp.float32),
            pltpu.VMEM((BPW,), jnp.float32),
            pltpu.SemaphoreType.DMA,
        ],
        compiler_params=pltpu.CompilerParams(needs_layout_passes=False,
                                             use_tc_tiling_on_sc=False),
    )
    def _combine(user_hbm, item_hbm, cat_hbm, su_hbm, si_hbm, sc_hbm,
                 bias_hbm, out_hbm,
                 uidx, iidx, cidx, su_v, si_v, sc_v, bias_vm, out_v, sem):
        wid = lax.axis_index("s") * NC + lax.axis_index("c")
        base = wid * BPW

        idx_copies = [
            pltpu.async_copy(user_hbm.at[wid], uidx, sem),
            pltpu.async_copy(item_hbm.at[wid], iidx, sem),
            pltpu.async_copy(cat_hbm.at[wid], cidx, sem),
            pltpu.async_copy(bias_hbm, bias_vm, sem),
        ]
        for cp in idx_copies:
            cp.wait()

        copies = []
        for j in range(NCHUNK):
            sl = pl.ds(j * CHUNK, CHUNK)
            copies.append(pltpu.async_copy(su_hbm.at[uidx.at[j]],
                                           su_v.at[sl], sem))
            copies.append(pltpu.async_copy(si_hbm.at[iidx.at[j]],
                                           si_v.at[sl], sem))
            copies.append(pltpu.async_copy(sc_hbm.at[cidx.at[j]],
                                           sc_v.at[sl], sem))
        for cp in copies:
            cp.wait()

        bias_v = bias_vm[...]

        def group_body(g, carry):
            sl = pl.ds(pl.multiple_of(g * L, L), L)
            logit = su_v[sl] + si_v[sl] + sc_v[sl] + bias_v
            out_v[sl] = 1.0 / (1.0 + jnp.exp(-logit))
            return carry

        lax.fori_loop(0, BPW // L, group_body, 0)

        pltpu.sync_copy(out_v, out_hbm.at[pl.ds(base, BPW)])

    return _combine


def kernel(user, item, category, user_table, item_table, category_table,
           fc_w, fc_b):
    user = user.astype(jnp.int32).reshape(NW, NCHUNK, CHUNK)
    item = item.astype(jnp.int32).reshape(NW, NCHUNK, CHUNK)
    category = category.astype(jnp.int32).reshape(NW, NCHUNK, CHUNK)
    w = fc_w.reshape(3 * D)
    s_u, s_i = _scores2(user_table.T, item_table.T, w[0:2 * D].reshape(2, D))
    s_c = _scores(category_table.T, w[2 * D:3 * D].reshape(1, D))
    bias = jnp.broadcast_to(fc_b.reshape(()), (L,))
    return _build_combine()(user, item, category, s_u, s_i, s_c, bias)


# speedup vs baseline: 1.0037x; 1.0037x over previous
import functools

import jax
import jax.numpy as jnp
from jax import lax
from jax.experimental import pallas as pl
from jax.experimental.pallas import tpu as pltpu
from jax.experimental.pallas import tpu_sc as plsc

NC = 2
NS = 16
L = 16
NW = NC * NS

B = 16384
D = 64
BPW = B // NW
CHUNK = 128
NCHUNK = BPW // CHUNK

MV_BLK = 16384


def _matvec_body(w_ref, t_ref, o_ref):
    o_ref[...] = lax.dot_general(
        w_ref[...], t_ref[...], (((1,), (0,)), ((), ())),
        preferred_element_type=jnp.float32).reshape(o_ref.shape)


def _scores(t_T, w_row):
    n = t_T.shape[1]
    blk = min(MV_BLK, n)
    grid = pl.cdiv(n, blk)
    out = pl.pallas_call(
        _matvec_body,
        grid=(grid,),
        in_specs=[
            pl.BlockSpec((1, D), lambda i: (0, 0)),
            pl.BlockSpec((D, blk), lambda i: (0, i)),
        ],
        out_specs=pl.BlockSpec((blk,), lambda i: (i,)),
        out_shape=jax.ShapeDtypeStruct((n,), jnp.float32),
    )(w_row, t_T)
    return out


def _matvec2_body(w_ref, tu_ref, ti_ref, ou_ref, oi_ref):
    ou_ref[...] = lax.dot_general(
        w_ref[0:1], tu_ref[...], (((1,), (0,)), ((), ())),
        preferred_element_type=jnp.float32).reshape(ou_ref.shape)
    oi_ref[...] = lax.dot_general(
        w_ref[1:2], ti_ref[...], (((1,), (0,)), ((), ())),
        preferred_element_type=jnp.float32).reshape(oi_ref.shape)


def _scores2(tu_T, ti_T, w2):
    n = tu_T.shape[1]
    blk = MV_BLK
    grid = pl.cdiv(n, blk)
    shape = jax.ShapeDtypeStruct((n,), jnp.float32)
    return pl.pallas_call(
        _matvec2_body,
        grid=(grid,),
        in_specs=[
            pl.BlockSpec((2, D), lambda i: (0, 0)),
            pl.BlockSpec((D, blk), lambda i: (0, i)),
            pl.BlockSpec((D, blk), lambda i: (0, i)),
        ],
        out_specs=[pl.BlockSpec((blk,), lambda i: (i,)),
                   pl.BlockSpec((blk,), lambda i: (i,))],
        out_shape=[shape, shape],
    )(w2, tu_T, ti_T)


@functools.cache
def _build_combine():
    mesh = plsc.VectorSubcoreMesh(core_axis_name="c", subcore_axis_name="s",
                                  num_cores=NC, num_subcores=NS)

    @functools.partial(
        pl.kernel,
        out_type=jax.ShapeDtypeStruct((B,), jnp.float32),
        mesh=mesh,
        scratch_types=[
            pltpu.VMEM((NCHUNK, CHUNK), jnp.int32),
            pltpu.VMEM((NCHUNK, CHUNK), jnp.int32),
            pltpu.VMEM((NCHUNK, CHUNK), jnp.int32),
            pltpu.VMEM((BPW,), jnp.float32),
            pltpu.VMEM((BPW,), jnp.float32),
            pltpu.VMEM((BPW,), jnp.float32),
            pltpu.VMEM((L,), jnp.float32),
            pltpu.VMEM((BPW,), jnp.float32),
            pltpu.SemaphoreType.DMA,
        ],
        compiler_params=pltpu.CompilerParams(needs_layout_passes=False,
                                             use_tc_tiling_on_sc=False),
    )
    def _combine(user_hbm, item_hbm, cat_hbm, su_hbm, si_hbm, sc_hbm,
                 bias_hbm, out_hbm,
                 uidx, iidx, cidx, su_v, si_v, sc_v, bias_vm, out_v, sem):
        wid = lax.axis_index("s") * NC + lax.axis_index("c")
        base = wid * BPW

        idx_copies = [
            pltpu.async_copy(user_hbm.at[wid], uidx, sem),
            pltpu.async_copy(item_hbm.at[wid], iidx, sem),
            pltpu.async_copy(cat_hbm.at[wid], cidx, sem),
            pltpu.async_copy(bias_hbm, bias_vm, sem),
        ]
        for cp in idx_copies:
            cp.wait()

        copies = []
        for j in range(NCHUNK):
            sl = pl.ds(j * CHUNK, CHUNK)
            copies.append(pltpu.async_copy(su_hbm.at[uidx.at[j]],
                                           su_v.at[sl], sem))
            copies.append(pltpu.async_copy(si_hbm.at[iidx.at[j]],
                                           si_v.at[sl], sem))
            copies.append(pltpu.async_copy(sc_hbm.at[cidx.at[j]],
                                           sc_v.at[sl], sem))
        for cp in copies:
            cp.wait()

        bias_v = bias_vm[...]

        def group_body(g, carry):
            sl = pl.ds(pl.multiple_of(g * L, L), L)
            logit = su_v[sl] + si_v[sl] + sc_v[sl] + bias_v
            out_v[sl] = 1.0 / (1.0 + jnp.exp(-logit))
            return carry

        lax.fori_loop(0, BPW // L, group_body, 0)

        pltpu.sync_copy(out_v, out_hbm.at[pl.ds(base, BPW)])

    return _combine


def kernel(user, item, category, user_table, item_table, category_table,
           fc_w, fc_b):
    user = user.astype(jnp.int32).reshape(NW, NCHUNK, CHUNK)
    item = item.astype(jnp.int32).reshape(NW, NCHUNK, CHUNK)
    category = category.astype(jnp.int32).reshape(NW, NCHUNK, CHUNK)
    w = fc_w.reshape(3 * D)
    s_u, s_i = _scores2(user_table.T, item_table.T, w[0:2 * D].reshape(2, D))
    s_c = _scores(category_table.T, w[2 * D:3 * D].reshape(1, D))
    bias = jnp.broadcast_to(fc_b.reshape(()), (L,))
    return _build_combine()(user, item, category, s_u, s_i, s_c, bias)
